# Initial kernel scaffold; baseline (speedup 1.0000x reference)
import jax
import jax.numpy as jnp
from jax.experimental import pallas as pl


def kernel(feat_p2, feat_p3, feat_p4, feat_p5, boxes_raw):
    raise NotImplementedError("write your pallas kernel here")


# speedup vs baseline: 37.2805x; 37.2805x over previous
import functools

import jax
import jax.numpy as jnp
from jax import lax
from jax.experimental import pallas as pl
from jax.experimental.pallas import tpu as pltpu
from jax.experimental.pallas import tpu_sc as plsc

OUT = 7
NBIN = OUT * OUT
NTERM = 16
K = NBIN * NTERM
N_ROI = 512
C = 256
H2 = 200
H3 = 100
P2_ROWS = 2 * H2 * H2
P3_ROWS = 2 * H3 * H3
T_ROWS = P2_ROWS + P3_ROWS
ROI_BLK = 64
CHUNK_BINS = 7
CHUNK_ROWS = CHUNK_BINS * NTERM
NCHUNK = NBIN // CHUNK_BINS
NC = 2
NS = 16
NW = NC * NS
ROI_PER_W = N_ROI // NW


def _term_geom(i, j, iy, ix, dy, dx, roi):
    (y1s, x1s, bin_h, bin_w, sf, si, base, bstride, b) = roi
    yf = y1s + (i.astype(jnp.float32) + (iy.astype(jnp.float32) * 0.5 + 0.25)) * bin_h
    xf = x1s + (j.astype(jnp.float32) + (ix.astype(jnp.float32) * 0.5 + 0.25)) * bin_w
    valid = (yf > -1.0) & (yf < sf) & (xf > -1.0) & (xf < sf)
    yc = jnp.clip(yf, 0.0, sf - 1.0)
    xc = jnp.clip(xf, 0.0, sf - 1.0)
    y0 = jnp.floor(yc)
    x0 = jnp.floor(xc)
    y0i = y0.astype(jnp.int32)
    x0i = x0.astype(jnp.int32)
    y1i = jnp.minimum(y0i + 1, si - 1)
    x1i = jnp.minimum(x0i + 1, si - 1)
    ly = yc - y0
    lx = xc - x0
    ycor = jnp.where(dy == 1, y1i, y0i)
    xcor = jnp.where(dx == 1, x1i, x0i)
    wy = jnp.where(dy == 1, ly, 1.0 - ly)
    wx = jnp.where(dx == 1, lx, 1.0 - lx)
    w = jnp.where(valid, 0.25 * (wy * wx), 0.0)
    idx = base + b * bstride + ycor * si + xcor
    idx = jnp.clip(idx, 0, T_ROWS - 1)
    return idx, w


def _roi_params(boxes_blk, n):
    u0 = boxes_blk[:, 0:1]
    u1 = boxes_blk[:, 1:2]
    u2 = boxes_blk[:, 2:3]
    u3 = boxes_blk[:, 3:4]
    x1 = u0 * 600.0
    y1 = u1 * 600.0
    x2 = x1 + 16.0 + u2 * 200.0
    y2 = y1 + 16.0 + u3 * 200.0
    area = (x2 - x1) * (y2 - y1)
    sizes = jnp.sqrt(area)
    lvl = jnp.floor(4.0 + jnp.log2(sizes / 224.0 + 1e-8))
    lvl = jnp.clip(lvl, 2.0, 5.0).astype(jnp.int32) - 2
    l = jnp.minimum(lvl, 1)
    is2 = l == 0
    scale = jnp.where(is2, 0.25, 0.125)
    sf = jnp.where(is2, 200.0, 100.0)
    si = jnp.where(is2, 200, 100)
    base = jnp.where(is2, 0, P2_ROWS)
    bstride = jnp.where(is2, H2 * H2, H3 * H3)
    b = n // 256
    x1s = x1 * scale - 0.5
    y1s = y1 * scale - 0.5
    x2s = x2 * scale - 0.5
    y2s = y2 * scale - 0.5
    bin_w = (x2s - x1s) / OUT
    bin_h = (y2s - y1s) / OUT
    return (y1s, x1s, bin_h, bin_w, sf, si, base, bstride, b)


def _prep_body(boxes_ref, idx_ref, w_ref):
    r = pl.program_id(0)
    nrow = r * ROI_BLK + lax.broadcasted_iota(jnp.int32, (ROI_BLK, 1), 0)
    roi = _roi_params(boxes_ref[...], nrow)

    k = lax.broadcasted_iota(jnp.int32, (ROI_BLK, K), 1)
    b2 = k // NTERM
    t2 = k % NTERM
    idx, _ = _term_geom(b2 // OUT, b2 % OUT, t2 // 8, (t2 // 4) % 2,
                        (t2 // 2) % 2, t2 % 2, roi)
    idx_ref[...] = idx

    roi3 = tuple(a[:, :, None] if a.ndim == 2 else a for a in roi)
    b3 = lax.broadcasted_iota(jnp.int32, (ROI_BLK, NBIN, C), 1)
    l3 = lax.broadcasted_iota(jnp.int32, (ROI_BLK, NBIN, C), 2)
    t3 = l3 // 16
    _, w = _term_geom(b3 // OUT, b3 % OUT, t3 // 8, (t3 // 4) % 2,
                      (t3 // 2) % 2, t3 % 2, roi3)
    w_ref[...] = w


def _prep(boxes, interpret=False):
    return pl.pallas_call(
        _prep_body,
        grid=(N_ROI // ROI_BLK,),
        in_specs=[pl.BlockSpec((ROI_BLK, 4), lambda r: (r, 0))],
        out_specs=[
            pl.BlockSpec((ROI_BLK, K), lambda r: (r, 0)),
            pl.BlockSpec((ROI_BLK, NBIN, C), lambda r: (r, 0, 0)),
        ],
        out_shape=[
            jax.ShapeDtypeStruct((N_ROI, K), jnp.int32),
            jax.ShapeDtypeStruct((N_ROI, NBIN, C), jnp.float32),
        ],
        interpret=interpret,
    )(boxes)


def _pack_p2_body(p2_ref, out_ref):
    out_ref[...] = p2_ref[0].reshape(C, 8 * H2).T


def _pack_p3_body(tacc_ref, p3_ref, out_ref):
    del tacc_ref
    out_ref[...] = p3_ref[0].reshape(128, H3 * H3).T


def _pack(feat_p2, feat_p3, interpret=False):
    t0 = pl.pallas_call(
        _pack_p2_body,
        grid=(50,),
        in_specs=[pl.BlockSpec((1, C, 8, H2), lambda g: (g // 25, 0, g % 25, 0))],
        out_specs=pl.BlockSpec((8 * H2, C), lambda g: (g, 0)),
        out_shape=jax.ShapeDtypeStruct((T_ROWS, C), jnp.float32),
        interpret=interpret,
    )(feat_p2)
    return pl.pallas_call(
        _pack_p3_body,
        grid=(2, 2),
        in_specs=[
            pl.BlockSpec(memory_space=pl.ANY),
            pl.BlockSpec((1, 128, H3, H3), lambda b, cb: (b, cb, 0, 0)),
        ],
        out_specs=pl.BlockSpec((H3 * H3, 128), lambda b, cb: (8 + b, cb)),
        out_shape=jax.ShapeDtypeStruct((T_ROWS, C), jnp.float32),
        input_output_aliases={0: 0},
        interpret=interpret,
    )(t0, feat_p3)


def _sc_body(t_hbm, idx_hbm, w_hbm, out_hbm, idxv, wv, ring0, ring1, ov,
             sem0, sem1):
    cid = lax.axis_index("c")
    sid = lax.axis_index("s")
    base_roi = (sid * NC + cid) * ROI_PER_W
    rings = (ring0, ring1)
    sems = (sem0, sem1)

    def roi_body(r, carry):
        n = base_roi + r
        pltpu.sync_copy(idx_hbm.at[n], idxv)
        pltpu.sync_copy(w_hbm.at[n], wv)

        def start(c):
            slot = c % 2
            return pltpu.async_copy(
                t_hbm.at[idxv.at[pl.ds(c * CHUNK_ROWS, CHUNK_ROWS)]],
                rings[slot], sems[slot])

        hs = [start(0), None]
        for c in range(NCHUNK):
            if c + 1 < NCHUNK:
                hs[(c + 1) % 2] = start(c + 1)
            hs[c % 2].wait()
            ringc = rings[c % 2]

            def bin_body(q, carry2, c=c, ringc=ringc):
                bi = c * CHUNK_BINS + q
                row0 = q * NTERM
                accs = [None] * 16
                for t in range(NTERM):
                    wt = wv[bi, pl.ds(t * 16, 16)]
                    for cc in range(16):
                        term = wt * ringc[row0 + t, pl.ds(cc * 16, 16)]
                        accs[cc] = term if accs[cc] is None else accs[cc] + term
                for cc in range(16):
                    ov[bi, pl.ds(cc * 16, 16)] = accs[cc]
                return carry2

            lax.fori_loop(0, CHUNK_BINS, bin_body, 0)
        pltpu.sync_copy(ov, out_hbm.at[n])
        return carry

    lax.fori_loop(0, ROI_PER_W, roi_body, 0)


def _sc_main(table, idx, w):
    mesh = plsc.VectorSubcoreMesh(core_axis_name="c", subcore_axis_name="s",
                                  num_cores=NC, num_subcores=NS)
    f = functools.partial(
        pl.kernel,
        out_type=jax.ShapeDtypeStruct((N_ROI, NBIN, C), jnp.float32),
        mesh=mesh,
        scratch_types=[
            pltpu.VMEM((K,), jnp.int32),
            pltpu.VMEM((NBIN, C), jnp.float32),
            pltpu.VMEM((CHUNK_ROWS, C), jnp.float32),
            pltpu.VMEM((CHUNK_ROWS, C), jnp.float32),
            pltpu.VMEM((NBIN, C), jnp.float32),
            pltpu.SemaphoreType.DMA,
            pltpu.SemaphoreType.DMA,
        ],
    )(_sc_body)
    return f(table, idx, w)


_FIN_BLK = 8


def _fin_body(in_ref, out_ref):
    out_ref[...] = jnp.transpose(in_ref[...], (0, 2, 1))


def _fin(outb, interpret=False):
    return pl.pallas_call(
        _fin_body,
        grid=(N_ROI // _FIN_BLK,),
        in_specs=[pl.BlockSpec((_FIN_BLK, NBIN, C), lambda r: (r, 0, 0))],
        out_specs=pl.BlockSpec((_FIN_BLK, C, NBIN), lambda r: (r, 0, 0)),
        out_shape=jax.ShapeDtypeStruct((N_ROI, C, NBIN), jnp.float32),
        interpret=interpret,
    )(outb)


def kernel(feat_p2, feat_p3, feat_p4, feat_p5, boxes_raw):
    del feat_p4, feat_p5
    boxes = boxes_raw.reshape(N_ROI, 4)
    idx, w = _prep(boxes)
    table = _pack(feat_p2, feat_p3)
    outb = _sc_main(table, idx, w)
    return _fin(outb).reshape(N_ROI, C, OUT, OUT)


# speedup vs baseline: 69.7333x; 1.8705x over previous
import functools

import jax
import jax.numpy as jnp
from jax import lax
from jax.experimental import pallas as pl
from jax.experimental.pallas import tpu as pltpu
from jax.experimental.pallas import tpu_sc as plsc

OUT = 7
NBIN = OUT * OUT
NTERM = 16
N_ROI = 512
C = 256
H2 = 200
H3 = 100
T2_ROWS = 2 * H2 * H2
T3_ROWS = 2 * H3 * H3
ROI_BLK = 64
CHUNK_BINS = OUT
CHUNK_ROWS = CHUNK_BINS * NTERM
NCHUNK = OUT
NC = 2
NS = 16
NW = NC * NS
ROI_PER_W = N_ROI // NW
NPAR = 8


def _prep_body(boxes_ref, par_ref):
    u0 = boxes_ref[:, 0:1]
    u1 = boxes_ref[:, 1:2]
    u2 = boxes_ref[:, 2:3]
    u3 = boxes_ref[:, 3:4]
    x1 = u0 * 600.0
    y1 = u1 * 600.0
    x2 = x1 + 16.0 + u2 * 200.0
    y2 = y1 + 16.0 + u3 * 200.0
    area = (x2 - x1) * (y2 - y1)
    sizes = jnp.sqrt(area)
    lvl = jnp.floor(4.0 + jnp.log2(sizes / 224.0 + 1e-8))
    lvl = jnp.clip(lvl, 2.0, 5.0).astype(jnp.int32) - 2
    is2 = lvl <= 0
    scale = jnp.where(is2, 0.25, 0.125)
    sf = jnp.where(is2, 200.0, 100.0)
    r = pl.program_id(0)
    n = r * ROI_BLK + lax.broadcasted_iota(jnp.int32, (ROI_BLK, 1), 0)
    b = (n // 256).astype(jnp.float32)
    base = b * jnp.where(is2, float(H2 * H2), float(H3 * H3))
    maxrow = jnp.where(is2, float(T2_ROWS - 1), float(T3_ROWS - 1))
    x1s = x1 * scale - 0.5
    y1s = y1 * scale - 0.5
    x2s = x2 * scale - 0.5
    y2s = y2 * scale - 0.5
    bin_w = (x2s - x1s) / OUT
    bin_h = (y2s - y1s) / OUT
    params = [y1s, x1s, bin_h, bin_w, sf, base,
              jnp.where(is2, 1.0, 0.0), maxrow]
    par_ref[...] = jnp.concatenate(
        [jnp.broadcast_to(p, (ROI_BLK, 16)) for p in params], axis=1)


def _prep(boxes, interpret=False):
    return pl.pallas_call(
        _prep_body,
        grid=(N_ROI // ROI_BLK,),
        in_specs=[pl.BlockSpec((ROI_BLK, 4), lambda r: (r, 0))],
        out_specs=pl.BlockSpec((ROI_BLK, NPAR * 16), lambda r: (r, 0)),
        out_shape=jax.ShapeDtypeStruct((N_ROI, NPAR * 16), jnp.float32),
        interpret=interpret,
    )(boxes)


def _sc_body(t2_hbm, t3_hbm, par_hbm, out_hbm, parv, ixbuf, ring0, ring1, ov,
             sem0, sem1):
    cid = lax.axis_index("c")
    sid = lax.axis_index("s")
    base_roi = (sid * NC + cid) * ROI_PER_W
    rings = (ring0, ring1)
    sems = (sem0, sem1)
    lv = lax.iota(jnp.int32, 16)
    iyf = (lv >> 3).astype(jnp.float32) * 0.5 + 0.25
    ixf = ((lv >> 2) & 1).astype(jnp.float32) * 0.5 + 0.25
    dyv = (lv >> 1) & 1
    dxv = lv & 1

    def roi_body(r, carry):
        n = base_roi + r
        pltpu.sync_copy(par_hbm.at[n], parv)
        y1s = parv[pl.ds(0, 16)]
        x1s = parv[pl.ds(16, 16)]
        bh = parv[pl.ds(32, 16)]
        bw = parv[pl.ds(48, 16)]
        sv = parv[pl.ds(64, 16)]
        basev = parv[pl.ds(80, 16)]
        is2v = parv[pl.ds(96, 16)]
        maxrv = parv[pl.ds(112, 16)]
        siv = sv.astype(jnp.int32)
        basei = basev.astype(jnp.int32)
        maxri = maxrv.astype(jnp.int32)
        is2 = is2v[0] > 0.5

        def fill_idx(c, slot):
            yf = y1s + (float(c) + iyf) * bh
            yc = jnp.clip(yf, 0.0, sv - 1.0)
            y0i = yc.astype(jnp.int32)
            ycor = jnp.where(dyv == 1, jnp.minimum(y0i + 1, siv - 1), y0i)
            rowy = basei + ycor * siv
            for j in range(CHUNK_BINS):
                xf = x1s + (float(j) + ixf) * bw
                xc = jnp.clip(xf, 0.0, sv - 1.0)
                x0i = xc.astype(jnp.int32)
                xcor = jnp.where(dxv == 1, jnp.minimum(x0i + 1, siv - 1), x0i)
                idx = jnp.clip(rowy + xcor, 0, maxri)
                ixbuf[slot, pl.ds(j * NTERM, NTERM)] = idx

        def start(slot):
            src_idx = ixbuf.at[slot]

            def go2():
                pltpu.async_copy(t2_hbm.at[src_idx], rings[slot], sems[slot])

            def go3():
                pltpu.async_copy(t3_hbm.at[src_idx], rings[slot], sems[slot])

            lax.cond(is2, go2, go3)

        def wait(slot):
            pltpu.make_async_copy(t2_hbm.at[ixbuf.at[slot]], rings[slot],
                                  sems[slot]).wait()

        fill_idx(0, 0)
        start(0)
        for c in range(NCHUNK):
            if c + 1 < NCHUNK:
                fill_idx(c + 1, (c + 1) % 2)
                start((c + 1) % 2)
            wait(c % 2)
            ringc = rings[c % 2]
            wys = []
            for iy in (0, 1):
                yf = y1s + (float(c) + (0.25 + 0.5 * iy)) * bh
                vy = (yf > -1.0) & (yf < sv)
                yc = jnp.clip(yf, 0.0, sv - 1.0)
                ly = yc - yc.astype(jnp.int32).astype(jnp.float32)
                wys.append((jnp.where(vy, 0.25 * (1.0 - ly), 0.0),
                            jnp.where(vy, 0.25 * ly, 0.0)))

            def bin_body(q, carry2, c=c, ringc=ringc, wys=wys):
                bi = c * CHUNK_BINS + q
                row0 = q * NTERM
                jf = jnp.broadcast_to(q, (16,)).astype(jnp.float32)
                wxs = []
                for ix in (0, 1):
                    xf = x1s + (jf + (0.25 + 0.5 * ix)) * bw
                    vx = (xf > -1.0) & (xf < sv)
                    xc = jnp.clip(xf, 0.0, sv - 1.0)
                    lx = xc - xc.astype(jnp.int32).astype(jnp.float32)
                    wxs.append((jnp.where(vx, 1.0 - lx, 0.0),
                                jnp.where(vx, lx, 0.0)))
                accs = [None] * 16
                for t in range(NTERM):
                    iy, ix, dy, dx = t // 8, (t // 4) % 2, (t // 2) % 2, t % 2
                    wt = wys[iy][dy] * wxs[ix][dx]
                    for cc in range(16):
                        term = wt * ringc[row0 + t, pl.ds(cc * 16, 16)]
                        accs[cc] = term if accs[cc] is None else accs[cc] + term
                for cc in range(16):
                    ov[bi, pl.ds(cc * 16, 16)] = accs[cc]
                return carry2

            lax.fori_loop(0, CHUNK_BINS, bin_body, 0)
        pltpu.sync_copy(ov, out_hbm.at[:, n, :])
        return carry

    lax.fori_loop(0, ROI_PER_W, roi_body, 0)


def _sc_main(t2, t3, par):
    mesh = plsc.VectorSubcoreMesh(core_axis_name="c", subcore_axis_name="s",
                                  num_cores=NC, num_subcores=NS)
    f = functools.partial(
        pl.kernel,
        out_type=jax.ShapeDtypeStruct((NBIN, N_ROI, C), jnp.float32),
        mesh=mesh,
        scratch_types=[
            pltpu.VMEM((NPAR * 16,), jnp.float32),
            pltpu.VMEM((2, CHUNK_ROWS), jnp.int32),
            pltpu.VMEM((CHUNK_ROWS, C), jnp.float32),
            pltpu.VMEM((CHUNK_ROWS, C), jnp.float32),
            pltpu.VMEM((NBIN, C), jnp.float32),
            pltpu.SemaphoreType.DMA,
            pltpu.SemaphoreType.DMA,
        ],
    )(_sc_body)
    return f(t2, t3, par)


def kernel(feat_p2, feat_p3, feat_p4, feat_p5, boxes_raw):
    del feat_p4, feat_p5
    boxes = boxes_raw.reshape(N_ROI, 4)
    par = _prep(boxes)
    t2 = feat_p2.transpose(0, 2, 3, 1).reshape(T2_ROWS, C)
    t3 = feat_p3.transpose(0, 2, 3, 1).reshape(T3_ROWS, C)
    out3 = _sc_main(t2, t3, par)
    return out3.transpose(1, 2, 0).reshape(N_ROI, C, OUT, OUT)


# speedup vs baseline: 73.2563x; 1.0505x over previous
import functools

import jax
import jax.numpy as jnp
from jax import lax
from jax.experimental import pallas as pl
from jax.experimental.pallas import tpu as pltpu
from jax.experimental.pallas import tpu_sc as plsc

OUT = 7
NBIN = OUT * OUT
NTERM = 16
N_ROI = 512
C = 256
H2 = 200
H3 = 100
T2_ROWS = 2 * H2 * H2
T3_ROWS = 2 * H3 * H3
ROI_BLK = 64
CHUNK_BINS = OUT
CHUNK_ROWS = CHUNK_BINS * NTERM
NCHUNK = OUT
NC = 2
NS = 16
NW = NC * NS
ROI_PER_W = N_ROI // NW
NPAR = 8


def _prep_body(boxes_ref, par_ref):
    u0 = boxes_ref[:, 0:1]
    u1 = boxes_ref[:, 1:2]
    u2 = boxes_ref[:, 2:3]
    u3 = boxes_ref[:, 3:4]
    x1 = u0 * 600.0
    y1 = u1 * 600.0
    x2 = x1 + 16.0 + u2 * 200.0
    y2 = y1 + 16.0 + u3 * 200.0
    area = (x2 - x1) * (y2 - y1)
    sizes = jnp.sqrt(area)
    lvl = jnp.floor(4.0 + jnp.log2(sizes / 224.0 + 1e-8))
    lvl = jnp.clip(lvl, 2.0, 5.0).astype(jnp.int32) - 2
    is2 = lvl <= 0
    scale = jnp.where(is2, 0.25, 0.125)
    sf = jnp.where(is2, 200.0, 100.0)
    r = pl.program_id(0)
    n = r * ROI_BLK + lax.broadcasted_iota(jnp.int32, (ROI_BLK, 1), 0)
    b = (n // 256).astype(jnp.float32)
    base = b * jnp.where(is2, float(H2 * H2), float(H3 * H3))
    maxrow = jnp.where(is2, float(T2_ROWS - 1), float(T3_ROWS - 1))
    x1s = x1 * scale - 0.5
    y1s = y1 * scale - 0.5
    x2s = x2 * scale - 0.5
    y2s = y2 * scale - 0.5
    bin_w = (x2s - x1s) / OUT
    bin_h = (y2s - y1s) / OUT
    params = [y1s, x1s, bin_h, bin_w, sf, base,
              jnp.where(is2, 1.0, 0.0), maxrow]
    par_ref[...] = jnp.concatenate(
        [jnp.broadcast_to(p, (ROI_BLK, 16)) for p in params], axis=1)


def _prep(boxes, interpret=False):
    return pl.pallas_call(
        _prep_body,
        grid=(N_ROI // ROI_BLK,),
        in_specs=[pl.BlockSpec((ROI_BLK, 4), lambda r: (r, 0))],
        out_specs=pl.BlockSpec((ROI_BLK, NPAR * 16), lambda r: (r, 0)),
        out_shape=jax.ShapeDtypeStruct((N_ROI, NPAR * 16), jnp.float32),
        interpret=interpret,
    )(boxes)


def _sc_body(t2_hbm, t3_hbm, par_hbm, out_hbm, parv, ixbuf, ring0, ring1, ov,
             sem0, sem1):
    cid = lax.axis_index("c")
    sid = lax.axis_index("s")
    base_roi = (sid * NC + cid) * ROI_PER_W
    rings = (ring0, ring1)
    sems = (sem0, sem1)
    lv = lax.iota(jnp.int32, 16)
    iyf = (lv >> 3).astype(jnp.float32) * 0.5 + 0.25
    ixf = ((lv >> 2) & 1).astype(jnp.float32) * 0.5 + 0.25
    dyv = (lv >> 1) & 1
    dxv = lv & 1

    def roi_body(r, carry):
        n = base_roi + r
        pltpu.sync_copy(par_hbm.at[n], parv)
        y1s = parv[pl.ds(0, 16)]
        x1s = parv[pl.ds(16, 16)]
        bh = parv[pl.ds(32, 16)]
        bw = parv[pl.ds(48, 16)]
        sv = parv[pl.ds(64, 16)]
        basev = parv[pl.ds(80, 16)]
        is2v = parv[pl.ds(96, 16)]
        maxrv = parv[pl.ds(112, 16)]
        siv = sv.astype(jnp.int32)
        basei = basev.astype(jnp.int32)
        maxri = maxrv.astype(jnp.int32)
        is2 = is2v[0] > 0.5

        def fill_idx(c, slot):
            yf = y1s + (float(c) + iyf) * bh
            yc = jnp.clip(yf, 0.0, sv - 1.0)
            y0i = yc.astype(jnp.int32)
            ycor = jnp.where(dyv == 1, jnp.minimum(y0i + 1, siv - 1), y0i)
            rowy = basei + ycor * siv
            for j in range(CHUNK_BINS):
                xf = x1s + (float(j) + ixf) * bw
                xc = jnp.clip(xf, 0.0, sv - 1.0)
                x0i = xc.astype(jnp.int32)
                xcor = jnp.where(dxv == 1, jnp.minimum(x0i + 1, siv - 1), x0i)
                idx = jnp.clip(rowy + xcor, 0, maxri)
                ixbuf[slot, pl.ds(j * NTERM, NTERM)] = idx

        def start(slot):
            src_idx = ixbuf.at[slot]

            def go2():
                pltpu.async_copy(t2_hbm.at[src_idx], rings[slot], sems[slot])

            def go3():
                pltpu.async_copy(t3_hbm.at[src_idx], rings[slot], sems[slot])

            lax.cond(is2, go2, go3)

        def wait(slot):
            pltpu.make_async_copy(t2_hbm.at[ixbuf.at[slot]], rings[slot],
                                  sems[slot]).wait()

        fill_idx(0, 0)
        start(0)
        for c in range(NCHUNK):
            if c + 1 < NCHUNK:
                fill_idx(c + 1, (c + 1) % 2)
                start((c + 1) % 2)
            wait(c % 2)
            ringc = rings[c % 2]
            wys = []
            for iy in (0, 1):
                yf = y1s + (float(c) + (0.25 + 0.5 * iy)) * bh
                vy = (yf > -1.0) & (yf < sv)
                yc = jnp.clip(yf, 0.0, sv - 1.0)
                ly = yc - yc.astype(jnp.int32).astype(jnp.float32)
                wys.append((jnp.where(vy, 0.25 * (1.0 - ly), 0.0),
                            jnp.where(vy, 0.25 * ly, 0.0)))

            def bin_body(q, carry2, c=c, ringc=ringc, wys=wys):
                bi = c * CHUNK_BINS + q
                row0 = q * NTERM
                jf = jnp.broadcast_to(q, (16,)).astype(jnp.float32)
                wxs = []
                for ix in (0, 1):
                    xf = x1s + (jf + (0.25 + 0.5 * ix)) * bw
                    vx = (xf > -1.0) & (xf < sv)
                    xc = jnp.clip(xf, 0.0, sv - 1.0)
                    lx = xc - xc.astype(jnp.int32).astype(jnp.float32)
                    wxs.append((jnp.where(vx, 1.0 - lx, 0.0),
                                jnp.where(vx, lx, 0.0)))
                accs = [None] * 16
                for t in range(NTERM):
                    iy, ix, dy, dx = t // 8, (t // 4) % 2, (t // 2) % 2, t % 2
                    wt = wys[iy][dy] * wxs[ix][dx]
                    for cc in range(16):
                        term = wt * ringc[row0 + t, pl.ds(cc * 16, 16)]
                        accs[cc] = term if accs[cc] is None else accs[cc] + term
                for cc in range(16):
                    ov[bi, pl.ds(cc * 16, 16)] = accs[cc]
                return carry2

            lax.fori_loop(0, CHUNK_BINS, bin_body, 0)
        pltpu.sync_copy(ov, out_hbm.at[:, n, :])
        return carry

    lax.fori_loop(0, ROI_PER_W, roi_body, 0)


def _sc_main(t2, t3, par):
    mesh = plsc.VectorSubcoreMesh(core_axis_name="c", subcore_axis_name="s",
                                  num_cores=NC, num_subcores=NS)
    f = functools.partial(
        pl.kernel,
        out_type=jax.ShapeDtypeStruct((NBIN, N_ROI, C), jnp.float32),
        mesh=mesh,
        scratch_types=[
            pltpu.VMEM((NPAR * 16,), jnp.float32),
            pltpu.VMEM((2, CHUNK_ROWS), jnp.int32),
            pltpu.VMEM((CHUNK_ROWS, C), jnp.float32),
            pltpu.VMEM((CHUNK_ROWS, C), jnp.float32),
            pltpu.VMEM((NBIN, C), jnp.float32),
            pltpu.SemaphoreType.DMA,
            pltpu.SemaphoreType.DMA,
        ],
    )(_sc_body)
    return f(t2, t3, par)


def kernel(feat_p2, feat_p3, feat_p4, feat_p5, boxes_raw):
    del feat_p4, feat_p5
    boxes = boxes_raw.reshape(N_ROI, 4)
    par = _prep(boxes)
    t2 = feat_p2.transpose(0, 2, 3, 1).reshape(T2_ROWS, C)
    t3 = feat_p3.reshape(2, C, H3 * H3).transpose(0, 2, 1).reshape(T3_ROWS, C)
    out3 = _sc_main(t2, t3, par)
    return out3.transpose(1, 2, 0).reshape(N_ROI, C, OUT, OUT)
